# Initial kernel scaffold; baseline (speedup 1.0000x reference)
import jax
import jax.numpy as jnp
from jax.experimental import pallas as pl


def kernel(pc, key):
    raise NotImplementedError("write your pallas kernel here")


# speedup vs baseline: 1.9758x; 1.9758x over previous
import functools

import jax
import jax.numpy as jnp
from jax import lax
from jax.experimental import pallas as pl
from jax.experimental.pallas import tpu as pltpu
from jax.experimental.pallas import tpu_sc as plsc

N_PTS = 65536
N_GRP = 1024
GRP_K = 32
ROWS = N_PTS // 128


def _fps_body(start_ref, xs_ref, ys_ref, zs_ref,
              idx_out, cx_out, cy_out, cz_out, dist_ref):
    dist_ref[...] = jnp.full((ROWS, 128), jnp.inf, dtype=jnp.float32)

    row_iota8 = lax.broadcasted_iota(jnp.int32, (8, 128), 0)
    lane_iota8 = lax.broadcasted_iota(jnp.int32, (8, 128), 1)
    lin = (lax.broadcasted_iota(jnp.int32, (ROWS, 128), 0) * 128
           + lax.broadcasted_iota(jnp.int32, (ROWS, 128), 1))

    def extract(ref, rb, mask):
        blk = ref[pl.ds(rb, 8), :]
        return jnp.sum(jnp.where(mask, blk, 0.0))

    def step(i, far):
        r = far // 128
        c = far % 128
        rb = (r // 8) * 8
        mask = (row_iota8 == (r - rb)) & (lane_iota8 == c)
        cx = extract(xs_ref, rb, mask)
        cy = extract(ys_ref, rb, mask)
        cz = extract(zs_ref, rb, mask)

        omask = (row_iota8 == (i // 128)) & (lane_iota8 == (i % 128))
        idx_out[...] = jnp.where(omask, far, idx_out[...])
        cx_out[...] = jnp.where(omask, cx, cx_out[...])
        cy_out[...] = jnp.where(omask, cy, cy_out[...])
        cz_out[...] = jnp.where(omask, cz, cz_out[...])

        dx = xs_ref[...] - cx
        dy = ys_ref[...] - cy
        dz = zs_ref[...] - cz
        d = (dx * dx + dz * dz) + dy * dy
        nd = jnp.minimum(dist_ref[...], d)
        dist_ref[...] = nd
        m = jnp.max(nd)
        cand = jnp.where(nd == m, lin, jnp.int32(N_PTS))
        return jnp.min(cand)

    lax.fori_loop(0, N_GRP, step, start_ref[0])


def _fps_pallas(xs, ys, zs, start):
    out_shapes = (
        jax.ShapeDtypeStruct((8, 128), jnp.int32),
        jax.ShapeDtypeStruct((8, 128), jnp.float32),
        jax.ShapeDtypeStruct((8, 128), jnp.float32),
        jax.ShapeDtypeStruct((8, 128), jnp.float32),
    )
    return pl.pallas_call(
        _fps_body,
        out_shape=out_shapes,
        in_specs=[
            pl.BlockSpec(memory_space=pltpu.SMEM),
            pl.BlockSpec(memory_space=pltpu.VMEM),
            pl.BlockSpec(memory_space=pltpu.VMEM),
            pl.BlockSpec(memory_space=pltpu.VMEM),
        ],
        out_specs=tuple(pl.BlockSpec(memory_space=pltpu.VMEM) for _ in range(4)),
        scratch_shapes=[pltpu.VMEM((ROWS, 128), jnp.float32)],
    )(start, xs, ys, zs)


_BN = 4096


def _dmat_body(q_ref, p_ref, q2_ref, r2_ref, o_ref):
    mm = lax.dot_general(q_ref[...], p_ref[...],
                         (((1,), (0,)), ((), ())),
                         preferred_element_type=jnp.float32)
    o_ref[...] = (q2_ref[...] - 2.0 * mm) + r2_ref[...]


def _dmat_pallas(q_pad, pcT_pad, q2, r2):
    grid = (N_PTS // _BN,)
    return pl.pallas_call(
        _dmat_body,
        out_shape=jax.ShapeDtypeStruct((N_GRP, N_PTS), jnp.float32),
        grid=grid,
        in_specs=[
            pl.BlockSpec((N_GRP, 8), lambda j: (0, 0)),
            pl.BlockSpec((8, _BN), lambda j: (0, j)),
            pl.BlockSpec((N_GRP, 1), lambda j: (0, 0)),
            pl.BlockSpec((1, _BN), lambda j: (0, j)),
        ],
        out_specs=pl.BlockSpec((N_GRP, _BN), lambda j: (0, j)),
    )(q_pad, pcT_pad, q2, r2)


_CHUNK = 16384
_NCHUNK = N_PTS // _CHUNK
_ROWS_PER_W = N_GRP // 32
_INF = float("inf")


def _merge16(xk, xi, yk, yi):
    yk_r = jnp.flip(yk)
    yi_r = jnp.flip(yi)
    c = xk <= yk_r
    lok = jnp.where(c, xk, yk_r)
    loi = jnp.where(c, xi, yi_r)
    hik = jnp.where(c, yk_r, xk)
    hii = jnp.where(c, yi_r, xi)
    lok, loi = plsc.sort_key_val(lok, loi)
    hik, hii = plsc.sort_key_val(hik, hii)
    return lok, loi, hik, hii


def _insert48(st, vk, vi):
    a0k, a0i, a1k, a1i, a2k, a2i, _t = st
    c0k, c0i, _, _ = _merge16(a2k, a2i, vk, vi)
    m1lk, m1li, m1hk, m1hi = _merge16(a1k, a1i, c0k, c0i)
    n0k, n0i, m2hk, m2hi = _merge16(a0k, a0i, m1lk, m1li)
    n1k, n1i, n2k, n2i = _merge16(m2hk, m2hi, m1hk, m1hi)
    nt = jnp.max(n1k)
    return (n0k, n0i, n1k, n1i, n2k, n2i, nt)


def _sc_topk_make(n_grp=N_GRP, n_pts=N_PTS, chunk=_CHUNK, interpret=False):
    nc, ns = 2, 16
    nw = nc * ns
    rows_per_w = n_grp // nw
    nchunk = n_pts // chunk
    mesh = plsc.VectorSubcoreMesh(
        core_axis_name="c", subcore_axis_name="s", num_cores=nc)

    @functools.partial(
        pl.kernel,
        mesh=mesh,
        out_type=jax.ShapeDtypeStruct((n_grp, GRP_K, 16), jnp.float32),
        scratch_types=[
            pltpu.VMEM((chunk,), jnp.float32),
            pltpu.VMEM((chunk,), jnp.float32),
            pltpu.VMEM((GRP_K,), jnp.int32),
            pltpu.VMEM((GRP_K, 16), jnp.float32),
            pltpu.VMEM((16,), jnp.float32),
            pltpu.SemaphoreType.DMA,
            pltpu.SemaphoreType.DMA,
            pltpu.SemaphoreType.DMA,
        ],
        compiler_params=pltpu.CompilerParams(
            needs_layout_passes=False, use_tc_tiling_on_sc=False),
        interpret=interpret,
    )
    def topk_kernel(d_hbm, pc_hbm, c_hbm, out_hbm,
                    buf0, buf1, idxb, rowsb, cbuf, sem0, sem1, gsem):
        wid = lax.axis_index("s") * nc + lax.axis_index("c")
        base = wid * rows_per_w
        lane = lax.iota(jnp.int32, 16)

        def do_row(rr, _):
            row = base + rr
            bufs = (buf0, buf1)
            sems = (sem0, sem1)

            inf16 = jnp.full((16,), _INF, dtype=jnp.float32)
            zero16 = jnp.zeros((16,), dtype=jnp.int32)
            st = (inf16, zero16, inf16, zero16, inf16, zero16,
                  jnp.full((), _INF, dtype=jnp.float32))

            cps = []
            cps.append(pltpu.async_copy(
                d_hbm.at[row, pl.ds(0, chunk)], bufs[0], sems[0]))
            for c in range(nchunk):
                if c + 1 < nchunk:
                    cps.append(pltpu.async_copy(
                        d_hbm.at[row, pl.ds((c + 1) * chunk, chunk)],
                        bufs[(c + 1) % 2], sems[(c + 1) % 2]))
                cps[c].wait()
                buf = bufs[c % 2]
                cbase = c * chunk

                def vstep(i, st_, buf=buf, cbase=cbase):
                    v = buf[pl.ds(i * 16, 16)]
                    m = v <= st_[6]
                    hit = jnp.any(m)

                    def ins(s):
                        vk = jnp.where(m, v, _INF)
                        vi = jnp.where(m, cbase + i * 16 + lane, 0)
                        vk, vi = plsc.sort_key_val(vk, vi)
                        return _insert48(s, vk, vi)

                    return lax.cond(hit, ins, lambda s: s, st_)

                st = lax.fori_loop(0, chunk // 16, vstep, st)

            k0, i0, k1, i1, k2, i2, _ = st
            win_lo = zero16
            win_hi = zero16
            big = jnp.int32(1 << 30)
            for k in range(GRP_K):
                m01 = jnp.minimum(k0, k1)
                mk = jnp.min(jnp.minimum(m01, k2))
                c0 = jnp.where(k0 == mk, i0, big)
                c1 = jnp.where(k1 == mk, i1, big)
                c2 = jnp.where(k2 == mk, i2, big)
                wi = jnp.min(jnp.minimum(jnp.minimum(c0, c1), c2))
                inv0 = (k0 == mk) & (i0 == wi)
                inv1 = (k1 == mk) & (i1 == wi)
                inv2 = (k2 == mk) & (i2 == wi)
                k0 = jnp.where(inv0, _INF, k0)
                k1 = jnp.where(inv1, _INF, k1)
                k2 = jnp.where(inv2, _INF, k2)
                if k < 16:
                    win_lo = jnp.where(lane == k, wi, win_lo)
                else:
                    win_hi = jnp.where(lane == (k - 16), wi, win_hi)

            idxb[pl.ds(0, 16)] = win_lo
            idxb[pl.ds(16, 16)] = win_hi

            pltpu.async_copy(pc_hbm.at[idxb], rowsb, gsem).wait()
            pltpu.sync_copy(c_hbm.at[row], cbuf)
            cv = cbuf[...]
            for j in range(GRP_K):
                rowsb[j] = rowsb[j] - cv
            pltpu.sync_copy(rowsb, out_hbm.at[row])
            return 0

        lax.fori_loop(0, rows_per_w, do_row, 0)

    return topk_kernel


def kernel(pc, key):
    rng = jax.random.key(key)
    start = jax.random.randint(rng, (), 0, N_PTS).astype(jnp.int32)[None]

    xs = pc[:, 0].reshape(ROWS, 128)
    ys = pc[:, 1].reshape(ROWS, 128)
    zs = pc[:, 2].reshape(ROWS, 128)

    idx8, cx8, cy8, cz8 = _fps_pallas(xs, ys, zs, start)
    center = jnp.stack(
        [cx8.reshape(-1), cy8.reshape(-1), cz8.reshape(-1)], axis=1)

    q2 = jnp.sum(center ** 2, axis=1, keepdims=True)
    r2 = jnp.sum(pc ** 2, axis=1)[None, :]
    q_pad = jnp.zeros((N_GRP, 8), jnp.float32).at[:, :3].set(center)
    pcT_pad = jnp.zeros((8, N_PTS), jnp.float32).at[:3, :].set(pc.T)

    d = _dmat_pallas(q_pad, pcT_pad, q2, r2)

    pc_pad = jnp.zeros((N_PTS, 16), jnp.float32).at[:, :3].set(pc)
    c_pad = jnp.zeros((N_GRP, 16), jnp.float32).at[:, :3].set(center)

    nb = _sc_topk_make()(d, pc_pad, c_pad)
    neighborhood = nb[:, :, :3]
    return (neighborhood, center)


# speedup vs baseline: 8.3518x; 4.2271x over previous
import functools

import jax
import jax.numpy as jnp
from jax import lax
from jax.experimental import pallas as pl
from jax.experimental.pallas import tpu as pltpu
from jax.experimental.pallas import tpu_sc as plsc

N_PTS = 65536
N_GRP = 1024
GRP_K = 32
ROWS = N_PTS // 128


def _fps_body(start_ref, xs_ref, ys_ref, zs_ref,
              idx_out, cx_out, cy_out, cz_out, dist_ref):
    dist_ref[...] = jnp.full((ROWS, 128), jnp.inf, dtype=jnp.float32)

    row_iota8 = lax.broadcasted_iota(jnp.int32, (8, 128), 0)
    lane_iota8 = lax.broadcasted_iota(jnp.int32, (8, 128), 1)
    lin = (lax.broadcasted_iota(jnp.int32, (ROWS, 128), 0) * 128
           + lax.broadcasted_iota(jnp.int32, (ROWS, 128), 1))

    def extract(ref, rb, mask):
        blk = ref[pl.ds(rb, 8), :]
        return jnp.sum(jnp.where(mask, blk, 0.0))

    def step(i, far):
        r = far // 128
        c = far % 128
        rb = (r // 8) * 8
        mask = (row_iota8 == (r - rb)) & (lane_iota8 == c)
        cx = extract(xs_ref, rb, mask)
        cy = extract(ys_ref, rb, mask)
        cz = extract(zs_ref, rb, mask)

        omask = (row_iota8 == (i // 128)) & (lane_iota8 == (i % 128))
        idx_out[...] = jnp.where(omask, far, idx_out[...])
        cx_out[...] = jnp.where(omask, cx, cx_out[...])
        cy_out[...] = jnp.where(omask, cy, cy_out[...])
        cz_out[...] = jnp.where(omask, cz, cz_out[...])

        dx = xs_ref[...] - cx
        dy = ys_ref[...] - cy
        dz = zs_ref[...] - cz
        d = (dx * dx + dz * dz) + dy * dy
        nd = jnp.minimum(dist_ref[...], d)
        dist_ref[...] = nd
        m = jnp.max(nd)
        cand = jnp.where(nd == m, lin, jnp.int32(N_PTS))
        return jnp.min(cand)

    lax.fori_loop(0, N_GRP, step, start_ref[0])


def _fps_pallas(xs, ys, zs, start):
    out_shapes = (
        jax.ShapeDtypeStruct((8, 128), jnp.int32),
        jax.ShapeDtypeStruct((8, 128), jnp.float32),
        jax.ShapeDtypeStruct((8, 128), jnp.float32),
        jax.ShapeDtypeStruct((8, 128), jnp.float32),
    )
    return pl.pallas_call(
        _fps_body,
        out_shape=out_shapes,
        in_specs=[
            pl.BlockSpec(memory_space=pltpu.SMEM),
            pl.BlockSpec(memory_space=pltpu.VMEM),
            pl.BlockSpec(memory_space=pltpu.VMEM),
            pl.BlockSpec(memory_space=pltpu.VMEM),
        ],
        out_specs=tuple(pl.BlockSpec(memory_space=pltpu.VMEM) for _ in range(4)),
        scratch_shapes=[pltpu.VMEM((ROWS, 128), jnp.float32)],
    )(start, xs, ys, zs)


_BN = 4096


def _dmat_body(q_ref, p_ref, q2_ref, r2_ref, o_ref):
    mm = lax.dot_general(q_ref[...], p_ref[...],
                         (((1,), (0,)), ((), ())),
                         preferred_element_type=jnp.float32)
    o_ref[...] = (q2_ref[...] - 2.0 * mm) + r2_ref[...]


def _dmat_pallas(q_pad, pcT_pad, q2, r2):
    grid = (N_PTS // _BN,)
    return pl.pallas_call(
        _dmat_body,
        out_shape=jax.ShapeDtypeStruct((N_GRP, N_PTS), jnp.float32),
        grid=grid,
        in_specs=[
            pl.BlockSpec((N_GRP, 8), lambda j: (0, 0)),
            pl.BlockSpec((8, _BN), lambda j: (0, j)),
            pl.BlockSpec((N_GRP, 1), lambda j: (0, 0)),
            pl.BlockSpec((1, _BN), lambda j: (0, j)),
        ],
        out_specs=pl.BlockSpec((N_GRP, _BN), lambda j: (0, j)),
    )(q_pad, pcT_pad, q2, r2)


_CHUNK = 16384
_NCHUNK = N_PTS // _CHUNK
_ROWS_PER_W = N_GRP // 32
_INF = float("inf")


def _merge16(xk, xi, yk, yi):
    yk_r = jnp.flip(yk)
    yi_r = jnp.flip(yi)
    c = xk <= yk_r
    lok = jnp.where(c, xk, yk_r)
    loi = jnp.where(c, xi, yi_r)
    hik = jnp.where(c, yk_r, xk)
    hii = jnp.where(c, yi_r, xi)
    lok, loi = plsc.sort_key_val(lok, loi)
    hik, hii = plsc.sort_key_val(hik, hii)
    return lok, loi, hik, hii


def _insert48(st, vk, vi):
    a0k, a0i, a1k, a1i, a2k, a2i, _t = st
    c0k, c0i, _, _ = _merge16(a2k, a2i, vk, vi)
    m1lk, m1li, m1hk, m1hi = _merge16(a1k, a1i, c0k, c0i)
    n0k, n0i, m2hk, m2hi = _merge16(a0k, a0i, m1lk, m1li)
    n1k, n1i, n2k, n2i = _merge16(m2hk, m2hi, m1hk, m1hi)
    nt = jnp.max(n1k)
    return (n0k, n0i, n1k, n1i, n2k, n2i, nt)


def _sc_topk_make(n_grp=N_GRP, n_pts=N_PTS, chunk=_CHUNK, interpret=False):
    nc, ns = 2, 16
    nw = nc * ns
    rows_per_w = n_grp // nw
    nchunk = n_pts // chunk
    mesh = plsc.VectorSubcoreMesh(
        core_axis_name="c", subcore_axis_name="s", num_cores=nc)

    @functools.partial(
        pl.kernel,
        mesh=mesh,
        out_type=jax.ShapeDtypeStruct((n_grp, GRP_K, 16), jnp.float32),
        scratch_types=[
            pltpu.VMEM((chunk,), jnp.float32),
            pltpu.VMEM((chunk,), jnp.float32),
            pltpu.VMEM((48,), jnp.float32),
            pltpu.VMEM((48,), jnp.int32),
            pltpu.SMEM((1,), jnp.float32),
            pltpu.VMEM((GRP_K,), jnp.int32),
            pltpu.VMEM((GRP_K, 16), jnp.float32),
            pltpu.VMEM((16,), jnp.float32),
            pltpu.SemaphoreType.DMA,
            pltpu.SemaphoreType.DMA,
            pltpu.SemaphoreType.DMA,
        ],
        compiler_params=pltpu.CompilerParams(
            needs_layout_passes=False, use_tc_tiling_on_sc=False),
        interpret=interpret,
    )
    def topk_kernel(d_hbm, pc_hbm, c_hbm, out_hbm,
                    buf0, buf1, kst, ist, tref, idxb, rowsb, cbuf,
                    sem0, sem1, gsem):
        wid = lax.axis_index("s") * nc + lax.axis_index("c")
        base = wid * rows_per_w
        lane = lax.iota(jnp.int32, 16)
        inf16 = jnp.full((16,), _INF, dtype=jnp.float32)
        zero16 = jnp.zeros((16,), dtype=jnp.int32)
        BATCH = 16

        def insert_vreg(v, m, vbase):
            vk = jnp.where(m, v, _INF)
            vi = jnp.where(m, vbase + lane, 0)
            vk, vi = plsc.sort_key_val(vk, vi)
            st = (kst[pl.ds(0, 16)], ist[pl.ds(0, 16)],
                  kst[pl.ds(16, 16)], ist[pl.ds(16, 16)],
                  kst[pl.ds(32, 16)], ist[pl.ds(32, 16)], None)
            n0k, n0i, n1k, n1i, n2k, n2i, nt = _insert48(st, vk, vi)
            kst[pl.ds(0, 16)] = n0k
            ist[pl.ds(0, 16)] = n0i
            kst[pl.ds(16, 16)] = n1k
            ist[pl.ds(16, 16)] = n1i
            kst[pl.ds(32, 16)] = n2k
            ist[pl.ds(32, 16)] = n2i
            tref[0] = nt

        def do_row(rr, _):
            row = base + rr
            bufs = (buf0, buf1)
            sems = (sem0, sem1)

            kst[pl.ds(0, 16)] = inf16
            kst[pl.ds(16, 16)] = inf16
            kst[pl.ds(32, 16)] = inf16
            ist[pl.ds(0, 16)] = zero16
            ist[pl.ds(16, 16)] = zero16
            ist[pl.ds(32, 16)] = zero16
            tref[0] = _INF

            cps = []
            cps.append(pltpu.async_copy(
                d_hbm.at[row, pl.ds(0, chunk)], bufs[0], sems[0]))
            for c in range(nchunk):
                if c + 1 < nchunk:
                    cps.append(pltpu.async_copy(
                        d_hbm.at[row, pl.ds((c + 1) * chunk, chunk)],
                        bufs[(c + 1) % 2], sems[(c + 1) % 2]))
                cps[c].wait()
                buf = bufs[c % 2]
                cbase = c * chunk

                def bstep(i, carry, buf=buf, cbase=cbase):
                    t = tref[0]
                    vs = [buf[pl.ds((i * BATCH + j) * 16, 16)]
                          for j in range(BATCH)]
                    tm = vs[0]
                    for j in range(1, BATCH):
                        tm = jnp.minimum(tm, vs[j])

                    def hit_batch():
                        for j in range(BATCH):
                            v = vs[j]
                            m = v <= tref[0]

                            def ins(v=v, m=m, j=j):
                                insert_vreg(v, m,
                                            cbase + (i * BATCH + j) * 16)

                            pl.when(jnp.any(m))(ins)

                    pl.when(jnp.any(tm <= t))(hit_batch)
                    return carry

                lax.fori_loop(0, chunk // (16 * BATCH), bstep, 0)

            k0 = kst[pl.ds(0, 16)]
            k1 = kst[pl.ds(16, 16)]
            k2 = kst[pl.ds(32, 16)]
            i0 = ist[pl.ds(0, 16)]
            i1 = ist[pl.ds(16, 16)]
            i2 = ist[pl.ds(32, 16)]
            win_lo = zero16
            win_hi = zero16
            big = jnp.int32(1 << 30)
            for k in range(GRP_K):
                m01 = jnp.minimum(k0, k1)
                mk = jnp.min(jnp.minimum(m01, k2))
                c0 = jnp.where(k0 == mk, i0, big)
                c1 = jnp.where(k1 == mk, i1, big)
                c2 = jnp.where(k2 == mk, i2, big)
                wi = jnp.min(jnp.minimum(jnp.minimum(c0, c1), c2))
                inv0 = (k0 == mk) & (i0 == wi)
                inv1 = (k1 == mk) & (i1 == wi)
                inv2 = (k2 == mk) & (i2 == wi)
                k0 = jnp.where(inv0, _INF, k0)
                k1 = jnp.where(inv1, _INF, k1)
                k2 = jnp.where(inv2, _INF, k2)
                if k < 16:
                    win_lo = jnp.where(lane == k, wi, win_lo)
                else:
                    win_hi = jnp.where(lane == (k - 16), wi, win_hi)

            idxb[pl.ds(0, 16)] = win_lo
            idxb[pl.ds(16, 16)] = win_hi

            pltpu.async_copy(pc_hbm.at[idxb], rowsb, gsem).wait()
            pltpu.sync_copy(c_hbm.at[row], cbuf)
            cv = cbuf[...]
            for j in range(GRP_K):
                rowsb[j] = rowsb[j] - cv
            pltpu.sync_copy(rowsb, out_hbm.at[row])
            return 0

        lax.fori_loop(0, rows_per_w, do_row, 0)

    return topk_kernel


def kernel(pc, key):
    rng = jax.random.key(key)
    start = jax.random.randint(rng, (), 0, N_PTS).astype(jnp.int32)[None]

    xs = pc[:, 0].reshape(ROWS, 128)
    ys = pc[:, 1].reshape(ROWS, 128)
    zs = pc[:, 2].reshape(ROWS, 128)

    idx8, cx8, cy8, cz8 = _fps_pallas(xs, ys, zs, start)
    center = jnp.stack(
        [cx8.reshape(-1), cy8.reshape(-1), cz8.reshape(-1)], axis=1)

    q2 = jnp.sum(center ** 2, axis=1, keepdims=True)
    r2 = jnp.sum(pc ** 2, axis=1)[None, :]
    q_pad = jnp.zeros((N_GRP, 8), jnp.float32).at[:, :3].set(center)
    pcT_pad = jnp.zeros((8, N_PTS), jnp.float32).at[:3, :].set(pc.T)

    d = _dmat_pallas(q_pad, pcT_pad, q2, r2)

    pc_pad = jnp.zeros((N_PTS, 16), jnp.float32).at[:, :3].set(pc)
    c_pad = jnp.zeros((N_GRP, 16), jnp.float32).at[:, :3].set(center)

    nb = _sc_topk_make()(d, pc_pad, c_pad)
    neighborhood = nb[:, :, :3]
    return (neighborhood, center)
